# baseline (speedup 1.0000x reference)
import functools

import jax
import jax.numpy as jnp
from jax import lax
from jax.experimental import pallas as pl
from jax.experimental.pallas import tpu as pltpu
from jax.experimental.pallas import tpu_sc as plsc

_CHUNK = 128


@functools.lru_cache(maxsize=None)
def _make_gather(n, d, chunk):
    info = plsc.get_sparse_core_info()
    nc, ns = info.num_cores, info.num_subcores
    nw = nc * ns
    assert n % (nw * chunk) == 0
    n_per_w = n // nw
    n_chunks = n_per_w // chunk
    mesh = plsc.VectorSubcoreMesh(core_axis_name="c", subcore_axis_name="s")

    @functools.partial(
        pl.kernel,
        mesh=mesh,
        out_type=jax.ShapeDtypeStruct((n, d), jnp.float32),
        compiler_params=pltpu.CompilerParams(use_tc_tiling_on_sc=False),
        scratch_types=[
            pltpu.VMEM((chunk,), jnp.int32),
            pltpu.VMEM((chunk, d), jnp.float32),
            pltpu.SemaphoreType.DMA,
        ],
    )
    def gather_kernel(table_hbm, idx_hbm, out_hbm, idx_v, rows_v, sem):
        wid = lax.axis_index("s") * nc + lax.axis_index("c")
        base = wid * n_per_w

        def body(i, carry):
            off = base + i * chunk
            pltpu.sync_copy(idx_hbm.at[pl.ds(off, chunk)], idx_v)
            pltpu.async_copy(table_hbm.at[idx_v], rows_v, sem).wait()
            pltpu.sync_copy(rows_v, out_hbm.at[pl.ds(off, chunk)])
            return carry

        lax.fori_loop(0, n_chunks, body, 0)

    return gather_kernel


def kernel(entity_cand_eid, table):
    b, m, k = entity_cand_eid.shape
    d = table.shape[1]
    n = b * m * k
    idx = entity_cand_eid.reshape(n)
    out = _make_gather(n, d, _CHUNK)(table, idx)
    return out.reshape(b, m, k, d)


# speedup vs baseline: 1.1410x; 1.1410x over previous
import functools

import jax
import jax.numpy as jnp
from jax import lax
from jax.experimental import pallas as pl
from jax.experimental.pallas import tpu as pltpu
from jax.experimental.pallas import tpu_sc as plsc

_IDXW = 128
_KSUB = 5
_BLK = _IDXW * _KSUB


@functools.lru_cache(maxsize=None)
def _make_gather(n, d):
    info = plsc.get_sparse_core_info()
    nc, ns = info.num_cores, info.num_subcores
    nw = nc * ns
    assert n % (nw * _BLK) == 0
    n_per_w = n // nw
    n_blocks = n_per_w // _BLK
    assert n_blocks % 2 == 0
    idx_rows_per_w = n_per_w // _IDXW
    mesh = plsc.VectorSubcoreMesh(core_axis_name="c", subcore_axis_name="s")

    @functools.partial(
        pl.kernel,
        mesh=mesh,
        out_type=jax.ShapeDtypeStruct((n, d), jnp.float32),
        compiler_params=pltpu.CompilerParams(use_tc_tiling_on_sc=False),
        scratch_types=[
            pltpu.VMEM((2, _KSUB, _IDXW), jnp.int32),
            pltpu.VMEM((2, _BLK, d), jnp.float32),
            pltpu.SemaphoreType.DMA,
            pltpu.SemaphoreType.DMA,
        ],
    )
    def gather_kernel(table_hbm, idx2d_hbm, out_hbm, idx_v, rows_v, sem_g, sem_out):
        wid = lax.axis_index("s") * nc + lax.axis_index("c")
        base = wid * n_per_w
        idx_row_base = wid * idx_rows_per_w

        def block_start(blk, buf):
            pltpu.sync_copy(
                idx2d_hbm.at[pl.ds(idx_row_base + blk * _KSUB, _KSUB)],
                idx_v.at[buf])
            return [
                pltpu.async_copy(
                    table_hbm.at[idx_v.at[buf, j]],
                    rows_v.at[buf, pl.ds(j * _IDXW, _IDXW)],
                    sem_g)
                for j in range(_KSUB)
            ]

        def store_start(blk, buf):
            pltpu.async_copy(
                rows_v.at[buf],
                out_hbm.at[pl.ds(base + blk * _BLK, _BLK)],
                sem_out)

        def store_wait(buf):
            pltpu.make_async_copy(
                rows_v.at[buf],
                out_hbm.at[pl.ds(base, _BLK)],
                sem_out).wait()

        def run_block(blk, buf):
            cps = block_start(blk, buf)
            for c in cps:
                c.wait()
            store_start(blk, buf)

        run_block(0, 0)
        run_block(1, 1)

        def body(g, carry):
            blk0 = 2 * g
            store_wait(0)
            run_block(blk0, 0)
            store_wait(1)
            run_block(blk0 + 1, 1)
            return carry

        lax.fori_loop(1, n_blocks // 2, body, 0)
        store_wait(0)
        store_wait(1)

    return gather_kernel


def kernel(entity_cand_eid, table):
    b, m, k = entity_cand_eid.shape
    d = table.shape[1]
    n = b * m * k
    idx2d = entity_cand_eid.reshape(n // _IDXW, _IDXW)
    out = _make_gather(n, d)(table, idx2d)
    return out.reshape(b, m, k, d)


# speedup vs baseline: 1.1475x; 1.0057x over previous
import functools

import jax
import jax.numpy as jnp
from jax import lax
from jax.experimental import pallas as pl
from jax.experimental.pallas import tpu as pltpu
from jax.experimental.pallas import tpu_sc as plsc

_IDXW = 128
_KSUB = 5
_BLK = _IDXW * _KSUB


@functools.lru_cache(maxsize=None)
def _make_gather(n, d):
    info = plsc.get_sparse_core_info()
    nc, ns = info.num_cores, info.num_subcores
    nw = nc * ns
    assert n % (nw * _BLK) == 0
    n_per_w = n // nw
    n_blocks = n_per_w // _BLK
    assert n_blocks % 2 == 0
    idx_rows_per_w = n_per_w // _IDXW
    mesh = plsc.VectorSubcoreMesh(core_axis_name="c", subcore_axis_name="s")

    @functools.partial(
        pl.kernel,
        mesh=mesh,
        out_type=jax.ShapeDtypeStruct((n, d), jnp.float32),
        compiler_params=pltpu.CompilerParams(use_tc_tiling_on_sc=False),
        scratch_types=[
            pltpu.VMEM((idx_rows_per_w, _IDXW), jnp.int32),
            pltpu.VMEM((2, _BLK, d), jnp.float32),
            pltpu.SemaphoreType.DMA,
            pltpu.SemaphoreType.DMA,
        ],
    )
    def gather_kernel(table_hbm, idx2d_hbm, out_hbm, idx_v, rows_v, sem_g, sem_out):
        wid = lax.axis_index("s") * nc + lax.axis_index("c")
        base = wid * n_per_w
        idx_row_base = wid * idx_rows_per_w

        pltpu.sync_copy(idx2d_hbm.at[pl.ds(idx_row_base, idx_rows_per_w)], idx_v)

        def block_start(blk, buf):
            return [
                pltpu.async_copy(
                    table_hbm.at[idx_v.at[blk * _KSUB + j]],
                    rows_v.at[buf, pl.ds(j * _IDXW, _IDXW)],
                    sem_g)
                for j in range(_KSUB)
            ]

        def store_start(blk, buf):
            pltpu.async_copy(
                rows_v.at[buf],
                out_hbm.at[pl.ds(base + blk * _BLK, _BLK)],
                sem_out)

        def store_wait(buf):
            pltpu.make_async_copy(
                rows_v.at[buf],
                out_hbm.at[pl.ds(base, _BLK)],
                sem_out).wait()

        def run_block(blk, buf):
            cps = block_start(blk, buf)
            for c in cps:
                c.wait()
            store_start(blk, buf)

        run_block(0, 0)
        run_block(1, 1)

        def body(g, carry):
            blk0 = 2 * g
            store_wait(0)
            run_block(blk0, 0)
            store_wait(1)
            run_block(blk0 + 1, 1)
            return carry

        lax.fori_loop(1, n_blocks // 2, body, 0)
        store_wait(0)
        store_wait(1)

    return gather_kernel


def kernel(entity_cand_eid, table):
    b, m, k = entity_cand_eid.shape
    d = table.shape[1]
    n = b * m * k
    idx2d = entity_cand_eid.reshape(n // _IDXW, _IDXW)
    out = _make_gather(n, d)(table, idx2d)
    return out.reshape(b, m, k, d)
